# Initial kernel scaffold; baseline (speedup 1.0000x reference)
import jax
import jax.numpy as jnp
from jax.experimental import pallas as pl


def kernel(src_code, logits):
    raise NotImplementedError("write your pallas kernel here")


# speedup vs baseline: 1.4098x; 1.4098x over previous
import jax
import jax.numpy as jnp
import numpy as np
from jax import lax
from jax.experimental import pallas as pl
from jax.experimental.pallas import tpu as pltpu

_ALPHABET = 1024
_BATCH = 16384
_R = 1024
_NBLK = _BATCH // _R

_K0 = np.uint32(0)
_K1 = np.uint32(42)
_K2 = np.uint32(0x1BD11BDA) ^ _K0 ^ _K1
_ROTS = ((13, 15, 26, 6), (17, 29, 16, 24))


def _rotl(x, d):
    return (x << np.uint32(d)) | (x >> np.uint32(32 - d))


def _threefry2x32_zero_hi(lin):
    ks = (_K0, _K1, _K2)
    x1 = lin + ks[1]
    x0 = x1
    x1 = x0 ^ _rotl(x1, _ROTS[0][0])
    for r in _ROTS[0][1:]:
        x0 = x0 + x1
        x1 = _rotl(x1, r)
        x1 = x0 ^ x1
    x0 = x0 + ks[1]
    x1 = x1 + ks[2] + np.uint32(1)
    for i in range(1, 5):
        for r in _ROTS[i % 2]:
            x0 = x0 + x1
            x1 = _rotl(x1, r)
            x1 = x0 ^ x1
        x0 = x0 + ks[(i + 1) % 3]
        x1 = x1 + ks[(i + 2) % 3] + np.uint32(i + 1)
    return x0, x1


_LN2 = np.float32(0.6931471805599453)


def _bits_to_gumbel(bits):
    fb = (bits >> np.uint32(9)) | np.uint32(0x3F800000)
    u = lax.bitcast_convert_type(fb, jnp.float32) - jnp.float32(1.0)
    u = u + jnp.float32(1e-10)
    return -jnp.log(-jnp.log(u))


def _body(idx_ref, table_ref, lin_ref, out_ref):
    i = pl.program_id(0)
    idx = idx_ref[0, 0]

    col = lax.broadcasted_iota(jnp.int32, (_R, _ALPHABET), 1)
    oh = jnp.where(idx[:, None] == col,
                   jnp.float32(1), jnp.float32(0)).astype(jnp.bfloat16)
    dn = (((1,), (0,)), ((), ()))
    rows = lax.dot_general(oh, table_ref[...], dn,
                           preferred_element_type=jnp.float32)

    base = jnp.uint32(i * (_R * _ALPHABET))
    lin = lin_ref[...] + base
    b0, b1 = _threefry2x32_zero_hi(lin)
    g = _bits_to_gumbel(b0 ^ b1)

    z = rows + g
    e = jnp.exp(z)
    s = jnp.sum(e, axis=1, keepdims=True)
    out_ref[...] = e * (jnp.float32(1.0) / s)


_LIN0 = np.uint32(_ALPHABET) * np.arange(_R, dtype=np.uint32)[:, None] \
    + np.arange(_ALPHABET, dtype=np.uint32)[None, :]


def _transducer(src_code, logits, interpret=False):
    idx3 = src_code.reshape(_NBLK, 1, _R)
    table_bf = logits.astype(jnp.bfloat16)
    out = pl.pallas_call(
        _body,
        grid=(_NBLK,),
        in_specs=[
            pl.BlockSpec((1, 1, _R), lambda i: (i, 0, 0)),
            pl.BlockSpec((_ALPHABET, _ALPHABET), lambda i: (0, 0)),
            pl.BlockSpec((_R, _ALPHABET), lambda i: (0, 0)),
        ],
        out_specs=pl.BlockSpec((_R, _ALPHABET), lambda i: (i, 0)),
        out_shape=jax.ShapeDtypeStruct((_BATCH, _ALPHABET), jnp.float32),
        interpret=interpret,
    )(idx3, table_bf, jnp.asarray(_LIN0))
    return out


def kernel(src_code, logits):
    return _transducer(src_code, logits)


# speedup vs baseline: 1.4455x; 1.0253x over previous
import functools

import jax
import jax.numpy as jnp
import numpy as np
from jax import lax
from jax.experimental import pallas as pl
from jax.experimental.pallas import tpu as pltpu
from jax.experimental.pallas import tpu_sc as plsc

_ALPHABET = 1024
_BATCH = 16384
_R = 1024
_A = 4096
_NBLK_LO = _A // _R
_NBLK_HI = (_BATCH - _A) // _R

_K0 = np.uint32(0)
_K1 = np.uint32(42)
_K2 = np.uint32(0x1BD11BDA) ^ _K0 ^ _K1
_ROTS = ((13, 15, 26, 6), (17, 29, 16, 24))


def _rotl(x, d):
    return (x << np.uint32(d)) | (x >> np.uint32(32 - d))


def _threefry2x32_zero_hi(lin):
    ks = (_K0, _K1, _K2)
    x1 = lin + ks[1]
    x0 = x1
    x1 = x0 ^ _rotl(x1, _ROTS[0][0])
    for r in _ROTS[0][1:]:
        x0 = x0 + x1
        x1 = _rotl(x1, r)
        x1 = x0 ^ x1
    x0 = x0 + ks[1]
    x1 = x1 + ks[2] + np.uint32(1)
    for i in range(1, 5):
        for r in _ROTS[i % 2]:
            x0 = x0 + x1
            x1 = _rotl(x1, r)
            x1 = x0 ^ x1
        x0 = x0 + ks[(i + 1) % 3]
        x1 = x1 + ks[(i + 2) % 3] + np.uint32(i + 1)
    return x0, x1


def _bits_to_gumbel(bits):
    fb = (bits >> np.uint32(9)) | np.uint32(0x3F800000)
    u = lax.bitcast_convert_type(fb, jnp.float32) - jnp.float32(1.0)
    u = u + jnp.float32(1e-10)
    return -jnp.log(-jnp.log(u))


_NW = 32
_RPW = _A // _NW
_CH = 8
_NCHUNK = _RPW // _CH
_VPC = _CH * _ALPHABET // 16


def _sc_bits_body(out_hbm, buf):
    wid = lax.axis_index("s") * 2 + lax.axis_index("c")
    base = wid * (_RPW * _ALPHABET)
    lane = lax.iota(jnp.uint32, 16)

    def chunk(ci, carry):
        chunk_base = base + ci * (_CH * _ALPHABET)

        def vec(j, carry2):
            lin = jnp.uint32(chunk_base + j * 16) + lane
            b0, b1 = _threefry2x32_zero_hi(lin)
            buf[pl.ds(j * 16, 16)] = b0 ^ b1
            return carry2

        lax.fori_loop(0, _VPC, vec, 0)
        pltpu.sync_copy(buf, out_hbm.at[pl.ds(chunk_base, _CH * _ALPHABET)])
        return carry

    lax.fori_loop(0, _NCHUNK, chunk, 0)


_sc_bits = functools.partial(
    pl.kernel,
    out_type=jax.ShapeDtypeStruct((_A * _ALPHABET,), jnp.uint32),
    mesh=plsc.VectorSubcoreMesh(core_axis_name="c", subcore_axis_name="s"),
    scratch_types=[pltpu.VMEM((_CH * _ALPHABET,), jnp.uint32)],
)(_sc_bits_body)


def _onehot_rows(idx, table_ref):
    col = lax.broadcasted_iota(jnp.int32, (_R, _ALPHABET), 1)
    oh = jnp.where(idx[:, None] == col,
                   jnp.float32(1), jnp.float32(0)).astype(jnp.bfloat16)
    dn = (((1,), (0,)), ((), ()))
    return lax.dot_general(oh, table_ref[...], dn,
                           preferred_element_type=jnp.float32)


def _softmax_store(rows, g, out_ref):
    e = jnp.exp(rows + g)
    s = jnp.sum(e, axis=1, keepdims=True)
    out_ref[...] = e * (jnp.float32(1.0) / s)


def _tc_rng_body(idx_ref, table_ref, lin_ref, out_ref):
    i = pl.program_id(0)
    rows = _onehot_rows(idx_ref[0, 0], table_ref)
    base = jnp.uint32((i + _NBLK_LO) * (_R * _ALPHABET))
    b0, b1 = _threefry2x32_zero_hi(lin_ref[...] + base)
    _softmax_store(rows, _bits_to_gumbel(b0 ^ b1), out_ref)


def _tc_bits_body(idx_ref, table_ref, bits_ref, out_ref):
    rows = _onehot_rows(idx_ref[0, 0], table_ref)
    _softmax_store(rows, _bits_to_gumbel(bits_ref[...]), out_ref)


_LIN0 = np.uint32(_ALPHABET) * np.arange(_R, dtype=np.uint32)[:, None] \
    + np.arange(_ALPHABET, dtype=np.uint32)[None, :]


def kernel(src_code, logits):
    table_bf = logits.astype(jnp.bfloat16)
    idx3 = src_code.reshape(_BATCH // _R, 1, _R)

    bits = _sc_bits().reshape(_A, _ALPHABET)

    y_full = pl.pallas_call(
        _tc_rng_body,
        grid=(_NBLK_HI,),
        in_specs=[
            pl.BlockSpec((1, 1, _R), lambda i: (i + _NBLK_LO, 0, 0)),
            pl.BlockSpec((_ALPHABET, _ALPHABET), lambda i: (0, 0)),
            pl.BlockSpec((_R, _ALPHABET), lambda i: (0, 0)),
        ],
        out_specs=pl.BlockSpec((_R, _ALPHABET), lambda i: (i + _NBLK_LO, 0)),
        out_shape=jax.ShapeDtypeStruct((_BATCH, _ALPHABET), jnp.float32),
    )(idx3, table_bf, jnp.asarray(_LIN0))

    y_lo = pl.pallas_call(
        _tc_bits_body,
        grid=(_NBLK_LO,),
        in_specs=[
            pl.BlockSpec((1, 1, _R), lambda i: (i, 0, 0)),
            pl.BlockSpec((_ALPHABET, _ALPHABET), lambda i: (0, 0)),
            pl.BlockSpec((_R, _ALPHABET), lambda i: (i, 0)),
        ],
        out_specs=pl.BlockSpec((_R, _ALPHABET), lambda i: (i, 0)),
        out_shape=jax.ShapeDtypeStruct((_A, _ALPHABET), jnp.float32),
    )(idx3, table_bf, bits)

    return lax.dynamic_update_slice(y_full, y_lo, (0, 0))
